# baseline (speedup 1.0000x reference)
import functools

import jax
import jax.numpy as jnp
from jax import lax
from jax.experimental import pallas as pl
from jax.experimental.pallas import tpu as pltpu
from jax.experimental.pallas import tpu_sc as plsc

M = 16
L = 16
NC, NS = 2, 16
NW = NC * NS
ROWS, COLS = 16384, 200
RT, CT = COLS, ROWS
SLAB_R = 8
SLAB_C = CT // NW
N_SLAB = RT // SLAB_R
NBUF = 4

_T_LO = 1e-6
_T_HI = 1.0 - 1e-6
_INV_DX = 1.0 / (1.0 / M + 1e-12)


def _build_tables(beta):
    babs = jnp.abs(beta)
    z = jnp.exp(-babs)
    y = z * (1.0 - z * (0.5 - z / 3.0))
    one_z = 1.0 + z
    for _ in range(4):
        y = y - 1.0 + one_z * jnp.exp(-y)
    sp = jnp.maximum(beta, 0.0) + y
    alpha = sp + 1e-8
    h = alpha / jnp.sum(alpha)
    y0 = plsc.cumsum(h) - h
    s = h * _INV_DX
    x0 = lax.iota(jnp.int32, M).astype(jnp.float32) * (1.0 / M)
    a = y0 - x0 * s
    return jnp.maximum(s, 1e-8), a


@functools.lru_cache(maxsize=1)
def _get_warp():
    mesh = plsc.VectorSubcoreMesh(
        core_axis_name="c", subcore_axis_name="s",
        num_cores=NC, num_subcores=NS)

    @functools.partial(
        pl.kernel,
        out_type=[jax.ShapeDtypeStruct((RT, CT), jnp.float32),
                  jax.ShapeDtypeStruct((RT, CT), jnp.float32)],
        mesh=mesh,
        compiler_params=pltpu.CompilerParams(needs_layout_passes=False),
        scratch_types=[
            pltpu.VMEM((M,), jnp.float32),
            pltpu.VMEM((M,), jnp.float32),
            pltpu.VMEM((NBUF, SLAB_R, SLAB_C), jnp.float32),
            pltpu.VMEM((NBUF, SLAB_R, SLAB_C), jnp.float32),
            pltpu.VMEM((NBUF, SLAB_R, SLAB_C), jnp.float32),
            pltpu.SemaphoreType.DMA((NBUF,)),
            pltpu.SemaphoreType.DMA((NBUF,)),
            pltpu.SemaphoreType.DMA((NBUF,)),
        ],
    )
    def _warp(u_hbm, beta_hbm, t_hbm, g_hbm,
              s_v, a_v, u_v, t_v, g_v, sem_u, sem_t, sem_g):
        cid = lax.axis_index("c")
        sid = lax.axis_index("s")
        wid = sid * NC + cid
        c0 = wid * SLAB_C

        def in_desc(k, b):
            return pltpu.make_async_copy(
                u_hbm.at[pl.ds(k * SLAB_R, SLAB_R), pl.ds(c0, SLAB_C)],
                u_v.at[b], sem_u.at[b])

        def out_t(k, b):
            return pltpu.make_async_copy(
                t_v.at[b], t_hbm.at[pl.ds(k * SLAB_R, SLAB_R), pl.ds(c0, SLAB_C)],
                sem_t.at[b])

        def out_g(k, b):
            return pltpu.make_async_copy(
                g_v.at[b], g_hbm.at[pl.ds(k * SLAB_R, SLAB_R), pl.ds(c0, SLAB_C)],
                sem_g.at[b])

        def _maybe(cond, fn):
            if isinstance(cond, bool):
                if cond:
                    fn()
            else:
                pl.when(cond)(fn)

        def step(kc, b):
            in_desc(kc, b).wait()

            def _wait_prev():
                out_t(kc - NBUF, b).wait()
                out_g(kc - NBUF, b).wait()

            _maybe(kc >= NBUF, _wait_prev)

            @plsc.parallel_loop(0, SLAB_R * SLAB_C, L, unroll=1)
            def _vec(off):
                r = lax.shift_right_logical(off, 9)
                c = lax.bitwise_and(off, SLAB_C - 1)
                uv = u_v[b, r, pl.ds(c, L)]
                idx = (uv * float(M)).astype(jnp.int32)
                sv = plsc.load_gather(s_v, [idx])
                av = plsc.load_gather(a_v, [idx])
                t = jnp.minimum(jnp.maximum(av + uv * sv, _T_LO), _T_HI)
                t_v[b, r, pl.ds(c, L)] = t
                g_v[b, r, pl.ds(c, L)] = sv

            out_t(kc, b).start()
            out_g(kc, b).start()
            _maybe(kc + NBUF < N_SLAB, lambda: in_desc(kc + NBUF, b).start())

        for b in range(NBUF):
            in_desc(b, b).start()

        pltpu.sync_copy(beta_hbm, s_v)
        s, a = _build_tables(s_v[...])
        s_v[...] = s
        a_v[...] = a

        @pl.loop(0, N_SLAB)
        def _outer(kc):
            step(kc, lax.rem(kc, NBUF))
        for kc in range(N_SLAB - NBUF, N_SLAB):
            out_t(kc, kc % NBUF).wait()
            out_g(kc, kc % NBUF).wait()

    return _warp


def kernel(u, beta):
    t, g = _get_warp()(u.T, beta)
    return (t.T, g.T)


# speedup vs baseline: 1.2466x; 1.2466x over previous
import functools

import jax
import jax.numpy as jnp
from jax import lax
from jax.experimental import pallas as pl
from jax.experimental.pallas import tpu as pltpu
from jax.experimental.pallas import tpu_sc as plsc

M = 16
L = 16
NC, NS = 2, 16
NW = NC * NS
ROWS, COLS = 16384, 200
RT, CT = COLS, ROWS
SLAB_R = 8
SLAB_C = CT // NW
N_SLAB = RT // SLAB_R
NBUF = 4

_T_LO = 1e-6
_T_HI = 1.0 - 1e-6
_INV_DX = 1.0 / (1.0 / M + 1e-12)


def _build_tables(beta):
    babs = jnp.abs(beta)
    z = jnp.exp(-babs)
    y = z * (1.0 - z * (0.5 - z / 3.0))
    one_z = 1.0 + z
    for _ in range(4):
        y = y - 1.0 + one_z * jnp.exp(-y)
    sp = jnp.maximum(beta, 0.0) + y
    alpha = sp + 1e-8
    h = alpha / jnp.sum(alpha)
    y0 = plsc.cumsum(h) - h
    s = h * _INV_DX
    x0 = lax.iota(jnp.int32, M).astype(jnp.float32) * (1.0 / M)
    a = y0 - x0 * s
    return jnp.maximum(s, 1e-8), a


@functools.lru_cache(maxsize=1)
def _get_warp():
    mesh = plsc.VectorSubcoreMesh(
        core_axis_name="c", subcore_axis_name="s",
        num_cores=NC, num_subcores=NS)

    @functools.partial(
        pl.kernel,
        out_type=[jax.ShapeDtypeStruct((RT, CT), jnp.float32),
                  jax.ShapeDtypeStruct((RT, CT), jnp.float32)],
        mesh=mesh,
        compiler_params=pltpu.CompilerParams(needs_layout_passes=False),
        scratch_types=[
            pltpu.VMEM((M,), jnp.float32),
            pltpu.VMEM((M,), jnp.float32),
            pltpu.VMEM((NBUF, SLAB_R, SLAB_C), jnp.float32),
            pltpu.VMEM((NBUF, SLAB_R, SLAB_C), jnp.float32),
            pltpu.VMEM((NBUF, SLAB_R, SLAB_C), jnp.float32),
            pltpu.SemaphoreType.DMA((NBUF,)),
            pltpu.SemaphoreType.DMA((NBUF,)),
            pltpu.SemaphoreType.DMA((NBUF,)),
        ],
    )
    def _warp(u_hbm, beta_hbm, t_hbm, g_hbm,
              s_v, a_v, u_v, t_v, g_v, sem_u, sem_t, sem_g):
        cid = lax.axis_index("c")
        sid = lax.axis_index("s")
        wid = sid * NC + cid
        c0 = wid * SLAB_C

        def in_desc(k, b):
            return pltpu.make_async_copy(
                u_hbm.at[pl.ds(k * SLAB_R, SLAB_R), pl.ds(c0, SLAB_C)],
                u_v.at[b], sem_u.at[b])

        def out_t(k, b):
            return pltpu.make_async_copy(
                t_v.at[b], t_hbm.at[pl.ds(k * SLAB_R, SLAB_R), pl.ds(c0, SLAB_C)],
                sem_t.at[b])

        def out_g(k, b):
            return pltpu.make_async_copy(
                g_v.at[b], g_hbm.at[pl.ds(k * SLAB_R, SLAB_R), pl.ds(c0, SLAB_C)],
                sem_g.at[b])

        def _maybe(cond, fn):
            if isinstance(cond, bool):
                if cond:
                    fn()
            else:
                pl.when(cond)(fn)

        def step(kc, b):
            in_desc(kc, b).wait()

            def _wait_prev():
                out_t(kc - NBUF, b).wait()
                out_g(kc - NBUF, b).wait()

            _maybe(kc >= NBUF, _wait_prev)

            @plsc.parallel_loop(0, SLAB_C, L, unroll=1)
            def _vec(off):
                for r in range(SLAB_R):
                    uv = u_v[b, r, pl.ds(off, L)]
                    idx = (uv * float(M)).astype(jnp.int32)
                    sv = plsc.load_gather(s_v, [idx])
                    av = plsc.load_gather(a_v, [idx])
                    t = jnp.minimum(jnp.maximum(av + uv * sv, _T_LO), _T_HI)
                    t_v[b, r, pl.ds(off, L)] = t
                    g_v[b, r, pl.ds(off, L)] = sv

            out_t(kc, b).start()
            out_g(kc, b).start()
            _maybe(kc + NBUF < N_SLAB, lambda: in_desc(kc + NBUF, b).start())

        for b in range(NBUF):
            in_desc(b, b).start()

        pltpu.sync_copy(beta_hbm, s_v)
        s, a = _build_tables(s_v[...])
        s_v[...] = s
        a_v[...] = a

        @pl.loop(0, N_SLAB)
        def _outer(kc):
            step(kc, lax.rem(kc, NBUF))
        for kc in range(N_SLAB - NBUF, N_SLAB):
            out_t(kc, kc % NBUF).wait()
            out_g(kc, kc % NBUF).wait()

    return _warp


def kernel(u, beta):
    t, g = _get_warp()(u.T, beta)
    return (t.T, g.T)


# speedup vs baseline: 1.2805x; 1.0272x over previous
import functools

import jax
import jax.numpy as jnp
from jax import lax
from jax.experimental import pallas as pl
from jax.experimental.pallas import tpu as pltpu
from jax.experimental.pallas import tpu_sc as plsc

M = 16
L = 16
NC, NS = 2, 16
NW = NC * NS
ROWS, COLS = 16384, 200
RT, CT = COLS, ROWS
SLAB_R = 8
SLAB_C = CT // NW
N_SLAB = RT // SLAB_R
NBUF = 4

_T_LO = 1e-6
_T_HI = 1.0 - 1e-6
_INV_DX = 1.0 / (1.0 / M + 1e-12)


def _build_tables(beta):
    babs = jnp.abs(beta)
    z = jnp.exp(-babs)
    y = z * (1.0 - z * (0.5 - z / 3.0))
    one_z = 1.0 + z
    for _ in range(4):
        y = y - 1.0 + one_z * jnp.exp(-y)
    sp = jnp.maximum(beta, 0.0) + y
    alpha = sp + 1e-8
    h = alpha / jnp.sum(alpha)
    y0 = plsc.cumsum(h) - h
    s = h * _INV_DX
    x0 = lax.iota(jnp.int32, M).astype(jnp.float32) * (1.0 / M)
    a = y0 - x0 * s
    return jnp.maximum(s, 1e-8), a


@functools.lru_cache(maxsize=1)
def _get_warp():
    mesh = plsc.VectorSubcoreMesh(
        core_axis_name="c", subcore_axis_name="s",
        num_cores=NC, num_subcores=NS)

    @functools.partial(
        pl.kernel,
        out_type=[jax.ShapeDtypeStruct((RT, CT), jnp.float32),
                  jax.ShapeDtypeStruct((RT, CT), jnp.float32)],
        mesh=mesh,
        compiler_params=pltpu.CompilerParams(needs_layout_passes=False),
        scratch_types=[
            pltpu.VMEM((M,), jnp.float32),
            pltpu.VMEM((M,), jnp.float32),
            pltpu.VMEM((NBUF, SLAB_R, SLAB_C), jnp.float32),
            pltpu.VMEM((NBUF, SLAB_R, SLAB_C), jnp.float32),
            pltpu.VMEM((NBUF, SLAB_R, SLAB_C), jnp.float32),
            pltpu.SemaphoreType.DMA((NBUF,)),
            pltpu.SemaphoreType.DMA((NBUF,)),
            pltpu.SemaphoreType.DMA((NBUF,)),
        ],
    )
    def _warp(u_hbm, beta_hbm, t_hbm, g_hbm,
              s_v, a_v, u_v, t_v, g_v, sem_u, sem_t, sem_g):
        cid = lax.axis_index("c")
        sid = lax.axis_index("s")
        wid = sid * NC + cid
        c0 = wid * SLAB_C

        def in_desc(k, b):
            return pltpu.make_async_copy(
                u_hbm.at[pl.ds(k * SLAB_R, SLAB_R), pl.ds(c0, SLAB_C)],
                u_v.at[b], sem_u.at[b])

        def out_t(k, b):
            return pltpu.make_async_copy(
                t_v.at[b], t_hbm.at[pl.ds(k * SLAB_R, SLAB_R), pl.ds(c0, SLAB_C)],
                sem_t.at[b])

        def out_g(k, b):
            return pltpu.make_async_copy(
                g_v.at[b], g_hbm.at[pl.ds(k * SLAB_R, SLAB_R), pl.ds(c0, SLAB_C)],
                sem_g.at[b])

        def _maybe(cond, fn):
            if isinstance(cond, bool):
                if cond:
                    fn()
            else:
                pl.when(cond)(fn)

        def step(kc, b):
            in_desc(kc, b).wait()

            def _wait_prev():
                out_t(kc - NBUF, b).wait()
                out_g(kc - NBUF, b).wait()

            _maybe(kc >= NBUF, _wait_prev)

            @plsc.parallel_loop(0, SLAB_C, L, unroll=2)
            def _vec(off):
                for r in range(SLAB_R):
                    uv = u_v[b, r, pl.ds(off, L)]
                    idx = (uv * float(M)).astype(jnp.int32)
                    sv = plsc.load_gather(s_v, [idx])
                    av = plsc.load_gather(a_v, [idx])
                    t = jnp.minimum(jnp.maximum(av + uv * sv, _T_LO), _T_HI)
                    t_v[b, r, pl.ds(off, L)] = t
                    g_v[b, r, pl.ds(off, L)] = sv

            out_t(kc, b).start()
            out_g(kc, b).start()
            _maybe(kc + NBUF < N_SLAB, lambda: in_desc(kc + NBUF, b).start())

        for b in range(NBUF):
            in_desc(b, b).start()

        pltpu.sync_copy(beta_hbm, s_v)
        s, a = _build_tables(s_v[...])
        s_v[...] = s
        a_v[...] = a

        @pl.loop(0, N_SLAB)
        def _outer(kc):
            step(kc, lax.rem(kc, NBUF))
        for kc in range(N_SLAB - NBUF, N_SLAB):
            out_t(kc, kc % NBUF).wait()
            out_g(kc, kc % NBUF).wait()

    return _warp


def kernel(u, beta):
    t, g = _get_warp()(u.T, beta)
    return (t.T, g.T)


# speedup vs baseline: 1.3115x; 1.0242x over previous
import functools

import jax
import jax.numpy as jnp
from jax import lax
from jax.experimental import pallas as pl
from jax.experimental.pallas import tpu as pltpu
from jax.experimental.pallas import tpu_sc as plsc

M = 16
L = 16
NC, NS = 2, 16
NW = NC * NS
ROWS, COLS = 16384, 200
RT, CT = COLS, ROWS
SLAB_R = 8
SLAB_C = CT // NW
N_SLAB = RT // SLAB_R
NBUF = 4

_T_LO = 1e-6
_T_HI = 1.0 - 1e-6
_INV_DX = 1.0 / (1.0 / M + 1e-12)


def _build_tables(beta):
    babs = jnp.abs(beta)
    z = jnp.exp(-babs)
    y = z * (1.0 - z * (0.5 - z / 3.0))
    one_z = 1.0 + z
    for _ in range(4):
        y = y - 1.0 + one_z * jnp.exp(-y)
    sp = jnp.maximum(beta, 0.0) + y
    alpha = sp + 1e-8
    h = alpha / jnp.sum(alpha)
    y0 = plsc.cumsum(h) - h
    s = h * _INV_DX
    x0 = lax.iota(jnp.int32, M).astype(jnp.float32) * (1.0 / M)
    a = y0 - x0 * s
    return jnp.maximum(s, 1e-8), a


@functools.lru_cache(maxsize=1)
def _get_warp():
    mesh = plsc.VectorSubcoreMesh(
        core_axis_name="c", subcore_axis_name="s",
        num_cores=NC, num_subcores=NS)

    @functools.partial(
        pl.kernel,
        out_type=[jax.ShapeDtypeStruct((RT, CT), jnp.float32),
                  jax.ShapeDtypeStruct((RT, CT), jnp.float32)],
        mesh=mesh,
        compiler_params=pltpu.CompilerParams(needs_layout_passes=False),
        scratch_types=[
            pltpu.VMEM((M,), jnp.float32),
            pltpu.VMEM((M,), jnp.float32),
            pltpu.VMEM((NBUF, SLAB_R, SLAB_C), jnp.float32),
            pltpu.VMEM((NBUF, SLAB_R, SLAB_C), jnp.float32),
            pltpu.VMEM((NBUF, SLAB_R, SLAB_C), jnp.float32),
            pltpu.SemaphoreType.DMA((NBUF,)),
            pltpu.SemaphoreType.DMA((NBUF,)),
            pltpu.SemaphoreType.DMA((NBUF,)),
        ],
    )
    def _warp(u_hbm, beta_hbm, t_hbm, g_hbm,
              s_v, a_v, u_v, t_v, g_v, sem_u, sem_t, sem_g):
        cid = lax.axis_index("c")
        sid = lax.axis_index("s")
        wid = sid * NC + cid
        c0 = wid * SLAB_C

        def in_desc(k, b):
            return pltpu.make_async_copy(
                u_hbm.at[pl.ds(k * SLAB_R, SLAB_R), pl.ds(c0, SLAB_C)],
                u_v.at[b], sem_u.at[b])

        def out_t(k, b):
            return pltpu.make_async_copy(
                t_v.at[b], t_hbm.at[pl.ds(k * SLAB_R, SLAB_R), pl.ds(c0, SLAB_C)],
                sem_t.at[b])

        def out_g(k, b):
            return pltpu.make_async_copy(
                g_v.at[b], g_hbm.at[pl.ds(k * SLAB_R, SLAB_R), pl.ds(c0, SLAB_C)],
                sem_g.at[b])

        def _maybe(cond, fn):
            if isinstance(cond, bool):
                if cond:
                    fn()
            else:
                pl.when(cond)(fn)

        def step(kc, b):
            in_desc(kc, b).wait()

            def _wait_prev():
                out_t(kc - NBUF, b).wait()
                out_g(kc - NBUF, b).wait()

            _maybe(kc >= NBUF, _wait_prev)

            @plsc.parallel_loop(0, SLAB_C, L, unroll=4)
            def _vec(off):
                for r in range(SLAB_R):
                    uv = u_v[b, r, pl.ds(off, L)]
                    idx = (uv * float(M)).astype(jnp.int32)
                    sv = plsc.load_gather(s_v, [idx])
                    av = plsc.load_gather(a_v, [idx])
                    t = jnp.minimum(jnp.maximum(av + uv * sv, _T_LO), _T_HI)
                    t_v[b, r, pl.ds(off, L)] = t
                    g_v[b, r, pl.ds(off, L)] = sv

            out_t(kc, b).start()
            out_g(kc, b).start()
            _maybe(kc + NBUF < N_SLAB, lambda: in_desc(kc + NBUF, b).start())

        for b in range(NBUF):
            in_desc(b, b).start()

        pltpu.sync_copy(beta_hbm, s_v)
        s, a = _build_tables(s_v[...])
        s_v[...] = s
        a_v[...] = a

        @pl.loop(0, N_SLAB)
        def _outer(kc):
            step(kc, lax.rem(kc, NBUF))
        for kc in range(N_SLAB - NBUF, N_SLAB):
            out_t(kc, kc % NBUF).wait()
            out_g(kc, kc % NBUF).wait()

    return _warp


def kernel(u, beta):
    t, g = _get_warp()(u.T, beta)
    return (t.T, g.T)
